# speedup vs baseline: 1.8953x; 1.8953x over previous
import jax
import jax.numpy as jnp
from jax import lax
from jax.experimental import pallas as pl

_BM = 2048


def _copy_body(x_ref, o_ref):
    i = pl.program_id(0)
    v = x_ref[...]
    rows = lax.broadcasted_iota(jnp.int32, v.shape, 0)
    cols = lax.broadcasted_iota(jnp.int32, v.shape, 1)
    hit = jnp.logical_and(i == 0,
                          jnp.logical_and(rows == 0, cols == 0))
    v = jnp.where(hit, jnp.float32(100.0), v)
    o_ref[0] = v
    o_ref[1] = v


def kernel(x):
    n, c = x.shape
    grid = (n // _BM,)
    out = pl.pallas_call(
        _copy_body,
        grid=grid,
        in_specs=[pl.BlockSpec((_BM, c), lambda i: (i, 0))],
        out_specs=pl.BlockSpec((2, _BM, c), lambda i: (0, i, 0)),
        out_shape=jax.ShapeDtypeStruct((2, n, c), x.dtype),
    )(x)
    return out.reshape(2 * n, c)
